# baseline (speedup 1.0000x reference)
import functools

import jax
import jax.numpy as jnp
from jax import lax
from jax.experimental import pallas as pl
from jax.experimental.pallas import tpu as pltpu
from jax.experimental.pallas import tpu_sc as plsc

NUM_CLASSES = 1000
FEAT_DIM = 512
LAMBDA_CENTER = 0.001
BATCH = 16384
LANES = 16
GROUPS_PER_ROW = FEAT_DIM // 32
NACC = 8

SC_ROWS = 11264
TC_ROWS = BATCH - SC_ROWS
NUM_WORKERS = 32
ROWS_PER_W = SC_ROWS // NUM_WORKERS
CHUNKS = (64, 64, 64, 64, 64, 32)
LCH = 64
NCHUNK = len(CHUNKS)

TC_BLK = 512


def _sc_partials(embeddings, labels3, centers_i32):
    mesh = plsc.VectorSubcoreMesh(core_axis_name="c", subcore_axis_name="s")

    @functools.partial(
        pl.kernel,
        mesh=mesh,
        out_type=jax.ShapeDtypeStruct((NUM_WORKERS, LANES), jnp.float32),
        compiler_params=pltpu.CompilerParams(needs_layout_passes=False),
        scratch_types=[
            pltpu.VMEM((NCHUNK * LCH,), jnp.int32),
            pltpu.VMEM((64, FEAT_DIM), jnp.float32),
            pltpu.VMEM((64, FEAT_DIM), jnp.float32),
            pltpu.VMEM((64, FEAT_DIM // 2), jnp.int32),
            pltpu.VMEM((64, FEAT_DIM // 2), jnp.int32),
            pltpu.VMEM((LANES,), jnp.float32),
            pltpu.SemaphoreType.DMA,
            pltpu.SemaphoreType.DMA,
            pltpu.SemaphoreType.DMA,
            pltpu.SemaphoreType.DMA,
        ],
    )
    def k(e_hbm, l_hbm, c_hbm, out_hbm, idx_v, eb0, eb1, cb0, cb1, accv,
          se0, se1, sc0, sc1):
        wid = lax.axis_index("s") * 2 + lax.axis_index("c")
        base = wid * ROWS_PER_W
        pltpu.sync_copy(l_hbm.at[pl.ds(base, NCHUNK * LCH)], idx_v)
        ebufs = (eb0, eb1)
        cbufs = (cb0, cb1)
        sems_e = (se0, se1)
        sems_c = (sc0, sc1)
        offs = [sum(CHUNKS[:i]) for i in range(NCHUNK)]

        def issue(ci):
            slot = ci % 2
            n = CHUNKS[ci]
            cpe = pltpu.async_copy(
                e_hbm.at[pl.ds(base + offs[ci], n)],
                ebufs[slot].at[pl.ds(0, n)], sems_e[slot])
            cpc = pltpu.async_copy(
                c_hbm.at[idx_v.at[pl.ds(offs[ci], n)]],
                cbufs[slot].at[pl.ds(0, n)], sems_c[slot])
            return cpe, cpc

        pending = issue(0)
        accs = tuple(jnp.zeros((LANES,), jnp.float32) for _ in range(NACC))
        for ci in range(NCHUNK):
            nxt = issue(ci + 1) if ci + 1 < NCHUNK else None
            pending[0].wait()
            pending[1].wait()
            slot = ci % 2
            eb = ebufs[slot]
            cb = cbufs[slot]

            def row_body(r, a, eb=eb, cb=cb):
                a = list(a)
                for g in range(GROUPS_PER_ROW):
                    c32i = cb[r, pl.ds(g * LANES, LANES)]
                    c32 = plsc.bitcast(c32i, jnp.bfloat16)
                    c_lo, c_hi = plsc.unpack(
                        c32, format=plsc.PackFormat.INTERLEAVED,
                        preferred_element_type=jnp.float32)
                    e_lo = eb[r, pl.ds(g * 32, LANES)]
                    e_hi = eb[r, pl.ds(g * 32 + LANES, LANES)]
                    d1 = e_lo - c_lo
                    d2 = e_hi - c_hi
                    a[(2 * g) % NACC] = a[(2 * g) % NACC] + d1 * d1
                    a[(2 * g + 1) % NACC] = a[(2 * g + 1) % NACC] + d2 * d2
                return tuple(a)

            accs = lax.fori_loop(0, CHUNKS[ci], row_body, accs)
            pending = nxt

        acc = accs[0]
        for i in range(1, NACC):
            acc = acc + accs[i]
        accv[...] = acc
        pltpu.sync_copy(accv, out_hbm.at[wid])

    return k(embeddings, labels3, centers_i32)


def _tc_partial(embeddings, labels2, centers):
    nblk = TC_ROWS // TC_BLK

    def body(e_hbm, l_ref, c_ref, o_ref, eb0, eb1, sem0, sem1):
        ebufs = (eb0, eb1)
        sems = (sem0, sem1)
        c_bf = c_ref[...].astype(jnp.bfloat16)

        def issue(b):
            slot = b % 2
            return pltpu.make_async_copy(
                e_hbm.at[pl.ds(SC_ROWS + b * TC_BLK, TC_BLK)],
                ebufs[slot], sems[slot]).start()

        issue(0)
        s = jnp.zeros((), jnp.float32)
        iot = lax.broadcasted_iota(jnp.int32, (TC_BLK, NUM_CLASSES), 1)
        for b in range(nblk):
            if b + 1 < nblk:
                issue(b + 1)
            slot = b % 2
            pltpu.make_async_copy(
                e_hbm.at[pl.ds(SC_ROWS + b * TC_BLK, TC_BLK)],
                ebufs[slot], sems[slot]).wait()
            lab = l_ref[pl.ds(SC_ROWS + b * TC_BLK, TC_BLK), :]
            oh = (lab == iot).astype(jnp.bfloat16)
            bc = jnp.dot(oh, c_bf, preferred_element_type=jnp.float32)
            d = ebufs[slot][...] - bc
            s = s + jnp.sum(d * d)
        o_ref[0, 0] = s

    return pl.pallas_call(
        body,
        in_specs=[
            pl.BlockSpec(memory_space=pl.ANY),
            pl.BlockSpec(memory_space=pltpu.VMEM),
            pl.BlockSpec(memory_space=pltpu.VMEM),
        ],
        out_specs=pl.BlockSpec(memory_space=pltpu.SMEM),
        out_shape=jax.ShapeDtypeStruct((1, 1), jnp.float32),
        scratch_shapes=[
            pltpu.VMEM((TC_BLK, FEAT_DIM), jnp.float32),
            pltpu.VMEM((TC_BLK, FEAT_DIM), jnp.float32),
            pltpu.SemaphoreType.DMA,
            pltpu.SemaphoreType.DMA,
        ],
    )(embeddings, labels2, centers)


def _finalize(partials, tc_part):
    def body(p_ref, t_ref, o_ref):
        o_ref[0, 0] = (jnp.sum(p_ref[...]) + t_ref[0, 0]) * (
            LAMBDA_CENTER / BATCH)

    out = pl.pallas_call(
        body,
        in_specs=[
            pl.BlockSpec(memory_space=pltpu.VMEM),
            pl.BlockSpec(memory_space=pltpu.SMEM),
        ],
        out_shape=jax.ShapeDtypeStruct((1, 1), jnp.float32),
        out_specs=pl.BlockSpec(memory_space=pltpu.SMEM),
    )(partials, tc_part)
    return out[0, 0]


def kernel(embeddings, labels, centers):
    labels_i = labels.astype(jnp.int32)
    labels2 = labels_i.reshape(BATCH, 1)
    x = centers.reshape(NUM_CLASSES, 16, 2, LANES)
    lo_u = lax.bitcast_convert_type(
        x[:, :, 0, :].astype(jnp.bfloat16), jnp.uint16).astype(jnp.uint32)
    hi_u = lax.bitcast_convert_type(
        x[:, :, 1, :].astype(jnp.bfloat16), jnp.uint16).astype(jnp.uint32)
    centers_i32 = lax.bitcast_convert_type(
        lo_u | (hi_u << 16), jnp.int32).reshape(NUM_CLASSES, FEAT_DIM // 2)
    partials = _sc_partials(embeddings, labels_i, centers_i32)
    tc_part = _tc_partial(embeddings, labels2, centers)
    return _finalize(partials, tc_part)


# speedup vs baseline: 1.0492x; 1.0492x over previous
import functools

import jax
import jax.numpy as jnp
from jax import lax
from jax.experimental import pallas as pl
from jax.experimental.pallas import tpu as pltpu
from jax.experimental.pallas import tpu_sc as plsc

NUM_CLASSES = 1000
FEAT_DIM = 512
LAMBDA_CENTER = 0.001
BATCH = 16384
LANES = 16
GROUPS_PER_ROW = FEAT_DIM // 32
NACC = 8

SC_ROWS = 11264
TC_ROWS = BATCH - SC_ROWS
NUM_WORKERS = 32
ROWS_PER_W = SC_ROWS // NUM_WORKERS
CHUNKS = (64, 64, 64, 64, 64, 32)
LCH = 64
NCHUNK = len(CHUNKS)

TC_BLK = 512


def _sc_partials(embeddings, labels3, centers_i32):
    mesh = plsc.VectorSubcoreMesh(core_axis_name="c", subcore_axis_name="s")

    @functools.partial(
        pl.kernel,
        mesh=mesh,
        out_type=jax.ShapeDtypeStruct((NUM_WORKERS, LANES), jnp.float32),
        compiler_params=pltpu.CompilerParams(needs_layout_passes=False),
        scratch_types=[
            pltpu.VMEM((NCHUNK, LCH), jnp.int32),
            pltpu.VMEM((64, FEAT_DIM), jnp.float32),
            pltpu.VMEM((64, FEAT_DIM), jnp.float32),
            pltpu.VMEM((64, FEAT_DIM // 2), jnp.int32),
            pltpu.VMEM((64, FEAT_DIM // 2), jnp.int32),
            pltpu.VMEM((LANES,), jnp.float32),
            pltpu.SemaphoreType.DMA,
            pltpu.SemaphoreType.DMA,
            pltpu.SemaphoreType.DMA,
            pltpu.SemaphoreType.DMA,
        ],
    )
    def k(e_hbm, l_hbm, c_hbm, out_hbm, idx_v, eb0, eb1, cb0, cb1, accv,
          se0, se1, sc0, sc1):
        wid = lax.axis_index("s") * 2 + lax.axis_index("c")
        base = wid * ROWS_PER_W
        pltpu.sync_copy(l_hbm.at[wid], idx_v)
        ebufs = (eb0, eb1)
        cbufs = (cb0, cb1)
        sems_e = (se0, se1)
        sems_c = (sc0, sc1)
        offs = [sum(CHUNKS[:i]) for i in range(NCHUNK)]

        def issue(ci):
            slot = ci % 2
            n = CHUNKS[ci]
            cpe = pltpu.async_copy(
                e_hbm.at[pl.ds(base + offs[ci], n)],
                ebufs[slot].at[pl.ds(0, n)], sems_e[slot])
            cpc = pltpu.async_copy(
                c_hbm.at[idx_v.at[ci, pl.ds(0, n)]],
                cbufs[slot].at[pl.ds(0, n)], sems_c[slot])
            return cpe, cpc

        pending = issue(0)
        accs = tuple(jnp.zeros((LANES,), jnp.float32) for _ in range(NACC))
        for ci in range(NCHUNK):
            nxt = issue(ci + 1) if ci + 1 < NCHUNK else None
            pending[0].wait()
            pending[1].wait()
            slot = ci % 2
            eb = ebufs[slot]
            cb = cbufs[slot]

            def row_body(r, a, eb=eb, cb=cb):
                a = list(a)
                for g in range(GROUPS_PER_ROW):
                    c32i = cb[r, pl.ds(g * LANES, LANES)]
                    c32 = plsc.bitcast(c32i, jnp.bfloat16)
                    c_lo, c_hi = plsc.unpack(
                        c32, format=plsc.PackFormat.INTERLEAVED,
                        preferred_element_type=jnp.float32)
                    e_lo = eb[r, pl.ds(g * 32, LANES)]
                    e_hi = eb[r, pl.ds(g * 32 + LANES, LANES)]
                    d1 = e_lo - c_lo
                    d2 = e_hi - c_hi
                    a[(2 * g) % NACC] = a[(2 * g) % NACC] + d1 * d1
                    a[(2 * g + 1) % NACC] = a[(2 * g + 1) % NACC] + d2 * d2
                return tuple(a)

            accs = lax.fori_loop(0, CHUNKS[ci], row_body, accs)
            pending = nxt

        acc = accs[0]
        for i in range(1, NACC):
            acc = acc + accs[i]
        accv[...] = acc
        pltpu.sync_copy(accv, out_hbm.at[wid])

    return k(embeddings, labels3, centers_i32)


def _tc_partial(embeddings, labels2, centers):
    nblk = TC_ROWS // TC_BLK

    def body(e_hbm, l_ref, c_ref, o_ref, eb0, eb1, sem0, sem1):
        ebufs = (eb0, eb1)
        sems = (sem0, sem1)
        c_bf = c_ref[...].astype(jnp.bfloat16)

        def issue(b):
            slot = b % 2
            return pltpu.make_async_copy(
                e_hbm.at[pl.ds(SC_ROWS + b * TC_BLK, TC_BLK)],
                ebufs[slot], sems[slot]).start()

        issue(0)
        s = jnp.zeros((), jnp.float32)
        iot = lax.broadcasted_iota(jnp.int32, (TC_BLK, NUM_CLASSES), 1)
        for b in range(nblk):
            if b + 1 < nblk:
                issue(b + 1)
            slot = b % 2
            pltpu.make_async_copy(
                e_hbm.at[pl.ds(SC_ROWS + b * TC_BLK, TC_BLK)],
                ebufs[slot], sems[slot]).wait()
            lab = l_ref[pl.ds(b * TC_BLK, TC_BLK), :]
            oh = (lab == iot).astype(jnp.bfloat16)
            bc = jnp.dot(oh, c_bf, preferred_element_type=jnp.float32)
            d = ebufs[slot][...] - bc
            s = s + jnp.sum(d * d)
        o_ref[0, 0] = s

    return pl.pallas_call(
        body,
        in_specs=[
            pl.BlockSpec(memory_space=pl.ANY),
            pl.BlockSpec(memory_space=pltpu.VMEM),
            pl.BlockSpec(memory_space=pltpu.VMEM),
        ],
        out_specs=pl.BlockSpec(memory_space=pltpu.SMEM),
        out_shape=jax.ShapeDtypeStruct((1, 1), jnp.float32),
        scratch_shapes=[
            pltpu.VMEM((TC_BLK, FEAT_DIM), jnp.float32),
            pltpu.VMEM((TC_BLK, FEAT_DIM), jnp.float32),
            pltpu.SemaphoreType.DMA,
            pltpu.SemaphoreType.DMA,
        ],
    )(embeddings, labels2, centers)


def _finalize(partials, tc_part):
    def body(p_ref, t_ref, o_ref):
        o_ref[0, 0] = (jnp.sum(p_ref[...]) + t_ref[0, 0]) * (
            LAMBDA_CENTER / BATCH)

    out = pl.pallas_call(
        body,
        in_specs=[
            pl.BlockSpec(memory_space=pltpu.VMEM),
            pl.BlockSpec(memory_space=pltpu.SMEM),
        ],
        out_shape=jax.ShapeDtypeStruct((1, 1), jnp.float32),
        out_specs=pl.BlockSpec(memory_space=pltpu.SMEM),
    )(partials, tc_part)
    return out[0, 0]


def kernel(embeddings, labels, centers):
    labels_i = labels.astype(jnp.int32)
    labels3 = jnp.pad(
        labels_i[:SC_ROWS].reshape(NUM_WORKERS, ROWS_PER_W),
        ((0, 0), (0, NCHUNK * LCH - ROWS_PER_W)),
    ).reshape(NUM_WORKERS, NCHUNK, LCH)
    labels2 = labels_i[SC_ROWS:].reshape(TC_ROWS, 1)
    x = centers.reshape(NUM_CLASSES, 16, 2, LANES)
    lo_u = lax.bitcast_convert_type(
        x[:, :, 0, :].astype(jnp.bfloat16), jnp.uint16).astype(jnp.uint32)
    hi_u = lax.bitcast_convert_type(
        x[:, :, 1, :].astype(jnp.bfloat16), jnp.uint16).astype(jnp.uint32)
    centers_i32 = lax.bitcast_convert_type(
        lo_u | (hi_u << 16), jnp.int32).reshape(NUM_CLASSES, FEAT_DIM // 2)
    partials = _sc_partials(embeddings, labels3, centers_i32)
    tc_part = _tc_partial(embeddings, labels2, centers)
    return _finalize(partials, tc_part)
